# Initial kernel scaffold; baseline (speedup 1.0000x reference)
import jax
import jax.numpy as jnp
from jax.experimental import pallas as pl


def kernel(x, edge_index, batch, params):
    raise NotImplementedError("write your pallas kernel here")


# speedup vs baseline: 1.5642x; 1.5642x over previous
import functools

import jax
import jax.numpy as jnp
from jax import lax
from jax.experimental import pallas as pl
from jax.experimental.pallas import tpu as pltpu
from jax.experimental.pallas import tpu_sc as plsc

_NC = 2
_NS = 16
_NW = _NC * _NS
_LANE = 16
_CB = 128


def _make_edge_pass(R, CH, D, U):
    per_sid = R // _NS
    n_seg = per_sid // _CB
    assert per_sid % _CB == 0 and CH % U == 0
    vec = D > 1
    out_shape = (_NC, R, D) if vec else (_NC, R)
    buf_shape = (_CB, D) if vec else (_CB,)
    acc_shape = (R, D) if vec else (R,)

    @functools.partial(
        pl.kernel,
        mesh=plsc.VectorSubcoreMesh(core_axis_name="c", subcore_axis_name="s"),
        out_type=jax.ShapeDtypeStruct(out_shape, jnp.float32),
        compiler_params=pltpu.CompilerParams(use_tc_tiling_on_sc=False),
        scratch_types=(
            [pltpu.VMEM((CH, _CB), jnp.int32),
             pltpu.VMEM((CH, _CB), jnp.int32)]
            + [pltpu.VMEM(buf_shape, jnp.float32) for _ in range(U)]
            + [pltpu.VMEM_SHARED(acc_shape, jnp.float32),
               pltpu.SemaphoreType.DMA]
        ),
    )
    def edge_pass(y_hbm, row_hbm, col_hbm, zeros_hbm, out_hbm, *refs):
        row_v, col_v = refs[0], refs[1]
        gbufs = refs[2:2 + U]
        acc = refs[2 + U]
        sem = refs[3 + U]
        cid = lax.axis_index("c")
        sid = lax.axis_index("s")
        wid = sid * _NC + cid

        pltpu.sync_copy(zeros_hbm, gbufs[0])

        def _fill(s, c):
            off = sid * per_sid + s * _CB
            pltpu.sync_copy(gbufs[0], acc.at[pl.ds(off, _CB)])
            return c
        lax.fori_loop(0, n_seg, _fill, 0)
        plsc.subcore_barrier()

        pltpu.sync_copy(row_hbm.at[wid], row_v)
        pltpu.sync_copy(col_hbm.at[wid], col_v)

        def _outer(jj, c):
            base = jj * U
            cps = []
            for u in range(U):
                cps.append(pltpu.async_copy(
                    y_hbm.at[row_v.at[base + u]], gbufs[u], sem))
            for u in range(U):
                cps[u].wait()
            for u in range(U):
                pltpu.sync_copy(gbufs[u], acc.at[col_v.at[base + u]],
                                add=True)
            return c
        lax.fori_loop(0, CH // U, _outer, 0)
        plsc.subcore_barrier()

        def _dump(s, c):
            off = sid * per_sid + s * _CB
            pltpu.sync_copy(acc.at[pl.ds(off, _CB)],
                            out_hbm.at[cid, pl.ds(off, _CB)])
            return c
        lax.fori_loop(0, n_seg, _dump, 0)

    return edge_pass


def _pad_idx(i, fill, CH):
    pad = _NW * CH * _CB - i.shape[0]
    ip = jnp.concatenate([i, jnp.full((pad,), fill, jnp.int32)])
    return ip.reshape(_NW, CH, _CB)


def kernel(x, edge_index, batch, params):
    n = x.shape[0]
    e = edge_index.shape[1]
    num_graphs = 64
    heads = 4
    sub = 32
    hidden = heads * sub
    ratio = 0.5
    seg = _NS * _CB
    R = ((n + 1 + seg - 1) // seg) * seg
    CH = ((-(-e // (_NW * _CB)) + 7) // 8) * 8
    half = hidden // 2
    scalar_pass = _make_edge_pass(R, CH, 1, 8)
    vec_pass = _make_edge_pass(R, CH, half, 4)
    zeros_s = jnp.zeros((_CB,), jnp.float32)
    zeros_v = jnp.zeros((_CB, half), jnp.float32)

    row = edge_index[0]
    col = edge_index[1]
    nm = jnp.ones((n,), bool)
    em = jnp.ones((e,), bool)
    x_cur = x
    reads = []
    for l in range(3):
        row_eff = jnp.where(em, row, n).astype(jnp.int32)
        col_eff = jnp.where(em, col, n).astype(jnp.int32)
        row3 = _pad_idx(row_eff, n, CH)
        col3 = _pad_idx(col_eff, n, CH)
        nmf = nm.astype(jnp.float32)

        y_deg = jnp.zeros((R,), jnp.float32).at[:n].set(nmf)
        dp = scalar_pass(y_deg, row3, col3, zeros_s)
        deg = dp[0, :n] + dp[1, :n] + nmf
        dinv = jnp.where(deg > 0, lax.rsqrt(deg), 0.0)

        Wcat = jnp.concatenate(
            [params['gcn%d_%d_W' % (l, i)] for i in range(heads)], axis=1)
        bcat = jnp.concatenate(
            [params['gcn%d_%d_b' % (l, i)] for i in range(heads)])
        xw = x_cur[:, :sub] @ Wcat
        y = jnp.zeros((R, hidden), jnp.float32).at[:n].set(dinv[:, None] * xw)
        ap0 = vec_pass(y[:, :half], row3, col3, zeros_v)
        ap1 = vec_pass(y[:, half:], row3, col3, zeros_v)
        acc = jnp.concatenate(
            [ap0[0, :n] + ap0[1, :n], ap1[0, :n] + ap1[1, :n]], axis=1)
        subs_cat = jax.nn.relu(
            dinv[:, None] * acc + (nmf * dinv * dinv)[:, None] * xw + bcat)

        sw = jnp.zeros((n,), jnp.float32)
        for i in range(heads):
            si = subs_cat[:, i * sub:(i + 1) * sub]
            a_i = si @ params['att%d_%d_w' % (l, i)]
            t_i = si @ params['score%d_W' % l][i * sub:(i + 1) * sub]
            sw = sw + (a_i * t_i)[:, 0]
        y_s = jnp.zeros((R,), jnp.float32).at[:n].set(dinv * sw)
        sp = scalar_pass(y_s, row3, col3, zeros_s)
        score = (dinv * (sp[0, :n] + sp[1, :n]) + nmf * dinv * dinv * sw
                 + params['score%d_b' % l][0])

        g = jnp.where(nm, batch, num_graphs).astype(jnp.int32)
        negs = jnp.where(nm, -score, jnp.inf)
        ii = jnp.arange(n, dtype=jnp.int32)
        g_s, _, idx_s = lax.sort((g, negs, ii), num_keys=2)
        cnt = jax.ops.segment_sum(jnp.ones((n,), jnp.int32), g,
                                  num_segments=num_graphs + 1)
        kk = jnp.ceil(ratio * cnt.astype(jnp.float32)).astype(jnp.int32)
        kk = kk.at[num_graphs].set(0)
        starts = jnp.cumsum(cnt) - cnt
        rank = jnp.arange(n, dtype=jnp.int32) - starts[g_s]
        sel_s = rank < kk[g_s]
        sel = jnp.zeros((n,), bool).at[idx_s].set(sel_s)

        em = em & sel[row] & sel[col]
        xx = subs_cat * jnp.tanh(score)[:, None]
        nm = sel
        bs = jnp.where(sel, batch, num_graphs)
        gm = jax.ops.segment_max(xx, bs, num_segments=num_graphs + 1)
        gm = gm[:num_graphs]
        csel = jax.ops.segment_sum(jnp.ones((n,), jnp.float32), bs,
                                   num_segments=num_graphs + 1)[:num_graphs]
        ga = (jax.ops.segment_sum(xx, bs, num_segments=num_graphs + 1)
              [:num_graphs] / jnp.maximum(csel, 1.0)[:, None])
        reads.append(jnp.concatenate([gm, ga], axis=1))
        x_cur = xx

    xo = reads[0] + reads[1] + reads[2]
    xo = jax.nn.relu(xo @ params['lin1_W'] + params['lin1_b'])
    xo = jax.nn.relu(xo @ params['lin2_W'] + params['lin2_b'])
    return jax.nn.log_softmax(xo @ params['lin3_W'] + params['lin3_b'],
                              axis=-1)


# speedup vs baseline: 1.5647x; 1.0003x over previous
import functools

import jax
import jax.numpy as jnp
from jax import lax
from jax.experimental import pallas as pl
from jax.experimental.pallas import tpu as pltpu
from jax.experimental.pallas import tpu_sc as plsc

_NC = 2
_NS = 16
_NW = _NC * _NS
_LANE = 16
_CB = 128


def _make_edge_pass(R, CH, D, U, CB):
    per_sid = R // _NS
    n_seg = per_sid // _CB
    assert per_sid % _CB == 0 and CH % U == 0
    vec = D > 1
    out_shape = (_NC, R, D) if vec else (_NC, R)
    buf_shape = (CB, D) if vec else (CB,)
    zbuf_shape = (_CB, D) if vec else (_CB,)
    acc_shape = (R, D) if vec else (R,)

    @functools.partial(
        pl.kernel,
        mesh=plsc.VectorSubcoreMesh(core_axis_name="c", subcore_axis_name="s"),
        out_type=jax.ShapeDtypeStruct(out_shape, jnp.float32),
        compiler_params=pltpu.CompilerParams(use_tc_tiling_on_sc=False),
        scratch_types=(
            [pltpu.VMEM((CH, CB), jnp.int32),
             pltpu.VMEM((CH, CB), jnp.int32),
             pltpu.VMEM(zbuf_shape, jnp.float32)]
            + [pltpu.VMEM(buf_shape, jnp.float32) for _ in range(U)]
            + [pltpu.VMEM_SHARED(acc_shape, jnp.float32),
               pltpu.SemaphoreType.DMA]
        ),
    )
    def edge_pass(y_hbm, row_hbm, col_hbm, zeros_hbm, out_hbm, *refs):
        row_v, col_v, zbuf = refs[0], refs[1], refs[2]
        gbufs = refs[3:3 + U]
        acc = refs[3 + U]
        sem = refs[4 + U]
        cid = lax.axis_index("c")
        sid = lax.axis_index("s")
        wid = sid * _NC + cid

        pltpu.sync_copy(zeros_hbm, zbuf)

        def _fill(s, c):
            off = sid * per_sid + s * _CB
            pltpu.sync_copy(zbuf, acc.at[pl.ds(off, _CB)])
            return c
        lax.fori_loop(0, n_seg, _fill, 0)
        plsc.subcore_barrier()

        pltpu.sync_copy(row_hbm.at[wid], row_v)
        pltpu.sync_copy(col_hbm.at[wid], col_v)

        def _outer(jj, c):
            base = jj * U
            cps = []
            for u in range(U):
                cps.append(pltpu.async_copy(
                    y_hbm.at[row_v.at[base + u]], gbufs[u], sem))
            for u in range(U):
                cps[u].wait()
            for u in range(U):
                pltpu.sync_copy(gbufs[u], acc.at[col_v.at[base + u]],
                                add=True)
            return c
        lax.fori_loop(0, CH // U, _outer, 0)
        plsc.subcore_barrier()

        def _dump(s, c):
            off = sid * per_sid + s * _CB
            pltpu.sync_copy(acc.at[pl.ds(off, _CB)],
                            out_hbm.at[cid, pl.ds(off, _CB)])
            return c
        lax.fori_loop(0, n_seg, _dump, 0)

    return edge_pass


def _pad_idx(i, fill, CH, CB):
    pad = _NW * CH * CB - i.shape[0]
    ip = jnp.concatenate([i, jnp.full((pad,), fill, jnp.int32)])
    return ip.reshape(_NW, CH, CB)


def kernel(x, edge_index, batch, params):
    n = x.shape[0]
    e = edge_index.shape[1]
    num_graphs = 64
    heads = 4
    sub = 32
    hidden = heads * sub
    ratio = 0.5
    seg = _NS * _CB
    R = ((n + 1 + seg - 1) // seg) * seg
    CB = 256
    U = 2
    CH = ((-(-e // (_NW * CB)) + U - 1) // U) * U
    half = hidden // 2
    scalar_pass = _make_edge_pass(R, CH, 1, U, CB)
    vec_pass = _make_edge_pass(R, CH, half, U, CB)
    zeros_s = jnp.zeros((_CB,), jnp.float32)
    zeros_v = jnp.zeros((_CB, half), jnp.float32)

    row = edge_index[0]
    col = edge_index[1]
    nm = jnp.ones((n,), bool)
    em = jnp.ones((e,), bool)
    x_cur = x
    reads = []
    for l in range(3):
        row_eff = jnp.where(em, row, n).astype(jnp.int32)
        col_eff = jnp.where(em, col, n).astype(jnp.int32)
        row3 = _pad_idx(row_eff, n, CH, CB)
        col3 = _pad_idx(col_eff, n, CH, CB)
        nmf = nm.astype(jnp.float32)

        y_deg = jnp.zeros((R,), jnp.float32).at[:n].set(nmf)
        dp = scalar_pass(y_deg, row3, col3, zeros_s)
        deg = dp[0, :n] + dp[1, :n] + nmf
        dinv = jnp.where(deg > 0, lax.rsqrt(deg), 0.0)

        Wcat = jnp.concatenate(
            [params['gcn%d_%d_W' % (l, i)] for i in range(heads)], axis=1)
        bcat = jnp.concatenate(
            [params['gcn%d_%d_b' % (l, i)] for i in range(heads)])
        xw = x_cur[:, :sub] @ Wcat
        y = jnp.zeros((R, hidden), jnp.float32).at[:n].set(dinv[:, None] * xw)
        ap0 = vec_pass(y[:, :half], row3, col3, zeros_v)
        ap1 = vec_pass(y[:, half:], row3, col3, zeros_v)
        acc = jnp.concatenate(
            [ap0[0, :n] + ap0[1, :n], ap1[0, :n] + ap1[1, :n]], axis=1)
        subs_cat = jax.nn.relu(
            dinv[:, None] * acc + (nmf * dinv * dinv)[:, None] * xw + bcat)

        sw = jnp.zeros((n,), jnp.float32)
        for i in range(heads):
            si = subs_cat[:, i * sub:(i + 1) * sub]
            a_i = si @ params['att%d_%d_w' % (l, i)]
            t_i = si @ params['score%d_W' % l][i * sub:(i + 1) * sub]
            sw = sw + (a_i * t_i)[:, 0]
        y_s = jnp.zeros((R,), jnp.float32).at[:n].set(dinv * sw)
        sp = scalar_pass(y_s, row3, col3, zeros_s)
        score = (dinv * (sp[0, :n] + sp[1, :n]) + nmf * dinv * dinv * sw
                 + params['score%d_b' % l][0])

        g = jnp.where(nm, batch, num_graphs).astype(jnp.int32)
        negs = jnp.where(nm, -score, jnp.inf)
        ii = jnp.arange(n, dtype=jnp.int32)
        g_s, _, idx_s = lax.sort((g, negs, ii), num_keys=2)
        cnt = jax.ops.segment_sum(jnp.ones((n,), jnp.int32), g,
                                  num_segments=num_graphs + 1)
        kk = jnp.ceil(ratio * cnt.astype(jnp.float32)).astype(jnp.int32)
        kk = kk.at[num_graphs].set(0)
        starts = jnp.cumsum(cnt) - cnt
        rank = jnp.arange(n, dtype=jnp.int32) - starts[g_s]
        sel_s = rank < kk[g_s]
        sel = jnp.zeros((n,), bool).at[idx_s].set(sel_s)

        em = em & sel[row] & sel[col]
        xx = subs_cat * jnp.tanh(score)[:, None]
        nm = sel
        bs = jnp.where(sel, batch, num_graphs)
        gm = jax.ops.segment_max(xx, bs, num_segments=num_graphs + 1)
        gm = gm[:num_graphs]
        csel = jax.ops.segment_sum(jnp.ones((n,), jnp.float32), bs,
                                   num_segments=num_graphs + 1)[:num_graphs]
        ga = (jax.ops.segment_sum(xx, bs, num_segments=num_graphs + 1)
              [:num_graphs] / jnp.maximum(csel, 1.0)[:, None])
        reads.append(jnp.concatenate([gm, ga], axis=1))
        x_cur = xx

    xo = reads[0] + reads[1] + reads[2]
    xo = jax.nn.relu(xo @ params['lin1_W'] + params['lin1_b'])
    xo = jax.nn.relu(xo @ params['lin2_W'] + params['lin2_b'])
    return jax.nn.log_softmax(xo @ params['lin3_W'] + params['lin3_b'],
                              axis=-1)


# speedup vs baseline: 5.9361x; 3.7938x over previous
import functools

import jax
import jax.numpy as jnp
from jax import lax
from jax.experimental import pallas as pl
from jax.experimental.pallas import tpu as pltpu
from jax.experimental.pallas import tpu_sc as plsc

_NC = 2
_NS = 16
_NW = _NC * _NS
_CB = 256
_BIT = 0x40000000


def _make_edge_pass(R, CH, D, const_src=False):
    per_sid = R // _NS
    assert R % _NS == 0
    FB = next(f for f in (1024, 512, 256, 128) if per_sid % f == 0)
    n_seg = per_sid // FB
    sent = R - 1
    vec = D > 1
    out_shape = (_NC, R, D) if vec else (_NC, R)
    buf_shape = (_CB, D) if vec else (_CB,)
    zbuf_shape = (FB, D) if vec else (FB,)
    acc_shape = (R, D) if vec else (R,)

    @functools.partial(
        pl.kernel,
        mesh=plsc.VectorSubcoreMesh(core_axis_name="c", subcore_axis_name="s"),
        out_type=jax.ShapeDtypeStruct(out_shape, jnp.float32),
        compiler_params=pltpu.CompilerParams(use_tc_tiling_on_sc=False),
        scratch_types=[
            pltpu.VMEM((CH + 1, _CB), jnp.int32),
            pltpu.VMEM((CH, _CB), jnp.int32),
            pltpu.VMEM(zbuf_shape, jnp.float32),
            pltpu.VMEM(buf_shape, jnp.float32),
            pltpu.VMEM_SHARED(acc_shape, jnp.float32),
            pltpu.SemaphoreType.DMA,
            pltpu.SemaphoreType.DMA,
        ],
    )
    def edge_pass(y_hbm, row_hbm, col_hbm, zeros_hbm, out_hbm,
                  row_v, col_v, zbuf, gbuf, acc, sem, sem2):
        cid = lax.axis_index("c")
        sid = lax.axis_index("s")
        wid = sid * _NC + cid

        pltpu.sync_copy(zeros_hbm, zbuf)

        def _fill(s, c):
            off = sid * per_sid + s * FB
            pltpu.sync_copy(zbuf, acc.at[pl.ds(off, FB)])
            return c
        lax.fori_loop(0, n_seg, _fill, 0)
        plsc.subcore_barrier()

        pltpu.sync_copy(row_hbm.at[wid], row_v.at[pl.ds(0, CH)])
        pltpu.sync_copy(col_hbm.at[wid], col_v)
        row_v[CH, pl.ds(0, 16)] = jnp.full((16,), sent, jnp.int32)
        if const_src:
            pltpu.sync_copy(y_hbm, gbuf)

        def _body(j, c):
            first = row_v[j, pl.ds(0, 16)][0]

            @pl.when(first != sent)
            def _():
                if not const_src:
                    pltpu.async_copy(
                        y_hbm.at[row_v.at[j]], gbuf, sem).wait()
                pltpu.async_copy(gbuf, acc.at[col_v.at[j]], sem2,
                                 add=True).wait()
            return c
        lax.fori_loop(0, CH, _body, 0)
        plsc.subcore_barrier()

        def _dump(s, c):
            off = sid * per_sid + s * FB
            pltpu.sync_copy(acc.at[pl.ds(off, FB)],
                            out_hbm.at[cid, pl.ds(off, FB)])
            return c
        lax.fori_loop(0, n_seg, _dump, 0)

    return edge_pass


def _interleave_pad(v, fill, CH):
    pad = _NW * CH * _CB - v.shape[0]
    vp = jnp.concatenate([v, jnp.full((pad,), fill, jnp.int32)])
    return vp.reshape(CH, _NW, _CB).transpose(1, 0, 2)


def kernel(x, edge_index, batch, params):
    n = x.shape[0]
    e = edge_index.shape[1]
    num_graphs = 64
    heads = 4
    sub = 32
    hidden = heads * sub
    half = hidden // 2
    ratio = 0.5

    seg = _NS * 128
    R = ((n + 1 + seg - 1) // seg) * seg
    sent = R - 1
    seg2 = _NS * 1024
    R2 = ((e + 1 + seg2 - 1) // seg2) * seg2
    sent2 = R2 - 1
    CHN = -(-e // (_NW * _CB))

    deg_pass = _make_edge_pass(R, CHN, 1, const_src=True)
    sc_pass = _make_edge_pass(R, CHN, 1)
    v_pass = _make_edge_pass(R, CHN, half)
    pk_pass = _make_edge_pass(R2, CHN, 1)
    zeros_s = jnp.zeros((128,), jnp.float32)
    zeros_v = jnp.zeros((128, half), jnp.float32)
    zeros_p = jnp.zeros((1024,), jnp.float32)
    ones_cb = jnp.ones((_CB,), jnp.float32)

    nm = jnp.ones((n,), bool)
    rowc = edge_index[0].astype(jnp.int32)
    colc = edge_index[1].astype(jnp.int32)
    m = jnp.int32(e)
    ii_e = jnp.arange(e, dtype=jnp.int32)
    x_cur = x
    reads = []
    for l in range(3):
        row3 = _interleave_pad(rowc, sent, CHN)
        col3 = _interleave_pad(colc, sent, CHN)
        nmf = nm.astype(jnp.float32)

        dp = deg_pass(ones_cb, row3, col3, zeros_s)
        deg = dp[0, :n] + dp[1, :n] + nmf
        dinv = jnp.where(deg > 0, lax.rsqrt(deg), 0.0)

        Wcat = jnp.concatenate(
            [params['gcn%d_%d_W' % (l, i)] for i in range(heads)], axis=1)
        bcat = jnp.concatenate(
            [params['gcn%d_%d_b' % (l, i)] for i in range(heads)])
        xw = x_cur[:, :sub] @ Wcat
        y = jnp.zeros((R, hidden), jnp.float32).at[:n].set(dinv[:, None] * xw)
        ap0 = v_pass(y[:, :half], row3, col3, zeros_v)
        ap1 = v_pass(y[:, half:], row3, col3, zeros_v)
        acc = jnp.concatenate(
            [ap0[0, :n] + ap0[1, :n], ap1[0, :n] + ap1[1, :n]], axis=1)
        subs_cat = jax.nn.relu(
            dinv[:, None] * acc + (nmf * dinv * dinv)[:, None] * xw + bcat)

        sw = jnp.zeros((n,), jnp.float32)
        for i in range(heads):
            si = subs_cat[:, i * sub:(i + 1) * sub]
            a_i = si @ params['att%d_%d_w' % (l, i)]
            t_i = si @ params['score%d_W' % l][i * sub:(i + 1) * sub]
            sw = sw + (a_i * t_i)[:, 0]
        y_s = jnp.zeros((R,), jnp.float32).at[:n].set(dinv * sw)
        sp = sc_pass(y_s, row3, col3, zeros_s)
        score = (dinv * (sp[0, :n] + sp[1, :n]) + nmf * dinv * dinv * sw
                 + params['score%d_b' % l][0])

        g = jnp.where(nm, batch, num_graphs).astype(jnp.int32)
        negs = jnp.where(nm, -score, jnp.inf)
        ii = jnp.arange(n, dtype=jnp.int32)
        g_s, _, idx_s = lax.sort((g, negs, ii), num_keys=2)
        cnt = jax.ops.segment_sum(jnp.ones((n,), jnp.int32), g,
                                  num_segments=num_graphs + 1)
        kk = jnp.ceil(ratio * cnt.astype(jnp.float32)).astype(jnp.int32)
        kk = kk.at[num_graphs].set(0)
        starts = jnp.cumsum(cnt) - cnt
        rank = jnp.arange(n, dtype=jnp.int32) - starts[g_s]
        sel_s = rank < kk[g_s]
        sel = jnp.zeros((n,), bool).at[idx_s].set(sel_s)

        xx = subs_cat * jnp.tanh(score)[:, None]
        bs = jnp.where(sel, batch, num_graphs)
        gm = jax.ops.segment_max(xx, bs, num_segments=num_graphs + 1)
        gm = gm[:num_graphs]
        csel = jax.ops.segment_sum(jnp.ones((n,), jnp.float32), bs,
                                   num_segments=num_graphs + 1)[:num_graphs]
        ga = (jax.ops.segment_sum(xx, bs, num_segments=num_graphs + 1)
              [:num_graphs] / jnp.maximum(csel, 1.0)[:, None])
        reads.append(jnp.concatenate([gm, ga], axis=1))
        x_cur = xx
        nm = sel

        if l < 2:
            alive = sel[rowc] & sel[colc] & (ii_e < m)
            m_new = jnp.sum(alive.astype(jnp.int32))
            pos = jnp.cumsum(alive.astype(jnp.int32)) - 1
            dest = jnp.where(alive, pos, sent2)
            gidx = jnp.where(ii_e < m, ii_e, sent2)
            g3 = _interleave_pad(gidx, sent2, CHN)
            d3 = _interleave_pad(dest, sent2, CHN)
            packed = []
            for v in (rowc, colc):
                yv = jnp.zeros((R2,), jnp.int32).at[:e].set(v | _BIT)
                yv = lax.bitcast_convert_type(yv, jnp.float32)
                pv = pk_pass(yv, g3, d3, zeros_p)
                b0 = lax.bitcast_convert_type(pv[0], jnp.int32)
                b1 = lax.bitcast_convert_type(pv[1], jnp.int32)
                packed.append(jnp.where(b0 != 0, b0, b1)[:e] & ~_BIT)
            rowc = jnp.where(ii_e < m_new, packed[0], sent)
            colc = jnp.where(ii_e < m_new, packed[1], sent)
            m = m_new

    xo = reads[0] + reads[1] + reads[2]
    xo = jax.nn.relu(xo @ params['lin1_W'] + params['lin1_b'])
    xo = jax.nn.relu(xo @ params['lin2_W'] + params['lin2_b'])
    return jax.nn.log_softmax(xo @ params['lin3_W'] + params['lin3_b'],
                              axis=-1)


# speedup vs baseline: 5.9367x; 1.0001x over previous
import functools

import jax
import jax.numpy as jnp
from jax import lax
from jax.experimental import pallas as pl
from jax.experimental.pallas import tpu as pltpu
from jax.experimental.pallas import tpu_sc as plsc

_NC = 2
_NS = 16
_NW = _NC * _NS
_CB = 256
_BIT = 0x40000000


def _make_edge_pass(R, CH, D, const_src=False):
    per_sid = R // _NS
    assert R % _NS == 0
    FB = next(f for f in (1024, 512, 256, 128) if per_sid % f == 0)
    n_seg = per_sid // FB
    sent = R - 1
    vec = D > 1
    out_shape = (_NC, R, D) if vec else (_NC, R)
    buf_shape = (_CB, D) if vec else (_CB,)
    zbuf_shape = (FB, D) if vec else (FB,)
    acc_shape = (R, D) if vec else (R,)

    @functools.partial(
        pl.kernel,
        mesh=plsc.VectorSubcoreMesh(core_axis_name="c", subcore_axis_name="s"),
        out_type=jax.ShapeDtypeStruct(out_shape, jnp.float32),
        compiler_params=pltpu.CompilerParams(use_tc_tiling_on_sc=False),
        scratch_types=[
            pltpu.VMEM((CH + 1, _CB), jnp.int32),
            pltpu.VMEM((CH, _CB), jnp.int32),
            pltpu.VMEM(zbuf_shape, jnp.float32),
            pltpu.VMEM(buf_shape, jnp.float32),
            pltpu.VMEM_SHARED(acc_shape, jnp.float32),
            pltpu.SemaphoreType.DMA,
            pltpu.SemaphoreType.DMA,
        ],
    )
    def edge_pass(y_hbm, row_hbm, col_hbm, zeros_hbm, out_hbm,
                  row_v, col_v, zbuf, gbuf, acc, sem, sem2):
        cid = lax.axis_index("c")
        sid = lax.axis_index("s")
        wid = sid * _NC + cid

        pltpu.sync_copy(zeros_hbm, zbuf)

        def _fill(s, c):
            off = sid * per_sid + s * FB
            pltpu.sync_copy(zbuf, acc.at[pl.ds(off, FB)])
            return c
        lax.fori_loop(0, n_seg, _fill, 0)
        plsc.subcore_barrier()

        pltpu.sync_copy(row_hbm.at[wid], row_v.at[pl.ds(0, CH)])
        pltpu.sync_copy(col_hbm.at[wid], col_v)
        row_v[CH, pl.ds(0, 16)] = jnp.full((16,), sent, jnp.int32)
        if const_src:
            pltpu.sync_copy(y_hbm, gbuf)

        def _body(j, c):
            first = row_v[j, pl.ds(0, 16)][0]

            @pl.when(first != sent)
            def _():
                if not const_src:
                    pltpu.async_copy(
                        y_hbm.at[row_v.at[j]], gbuf, sem).wait()
                pltpu.async_copy(gbuf, acc.at[col_v.at[j]], sem2,
                                 add=True).wait()
            return c
        lax.fori_loop(0, CH, _body, 0)
        plsc.subcore_barrier()

        def _dump(s, c):
            off = sid * per_sid + s * FB
            pltpu.sync_copy(acc.at[pl.ds(off, FB)],
                            out_hbm.at[cid, pl.ds(off, FB)])
            return c
        lax.fori_loop(0, n_seg, _dump, 0)

    return edge_pass


def _interleave_pad(v, fill, CH):
    pad = _NW * CH * _CB - v.shape[0]
    vp = jnp.concatenate([v, jnp.full((pad,), fill, jnp.int32)])
    return vp.reshape(CH, _NW, _CB).transpose(1, 0, 2)


def kernel(x, edge_index, batch, params):
    n = x.shape[0]
    e = edge_index.shape[1]
    num_graphs = 64
    heads = 4
    sub = 32
    hidden = heads * sub
    half = hidden // 2
    ratio = 0.5

    seg = _NS * 128
    R = ((n + 1 + seg - 1) // seg) * seg
    sent = R - 1
    seg2 = _NS * 1024
    R2 = ((e + 1 + seg2 - 1) // seg2) * seg2
    sent2 = R2 - 1
    CHN = -(-e // (_NW * _CB))

    deg_pass = _make_edge_pass(R, CHN, 1, const_src=True)
    sc_pass = _make_edge_pass(R, CHN, 1)
    v_pass = _make_edge_pass(R, CHN, half)
    pk_pass = _make_edge_pass(R2, CHN, 1)
    zeros_s = jnp.zeros((128,), jnp.float32)
    zeros_v = jnp.zeros((128, half), jnp.float32)
    zeros_p = jnp.zeros((1024,), jnp.float32)
    ones_cb = jnp.ones((_CB,), jnp.float32)

    nm = jnp.ones((n,), bool)
    rowc = edge_index[0].astype(jnp.int32)
    colc = edge_index[1].astype(jnp.int32)
    m = jnp.int32(e)
    ii_e = jnp.arange(e, dtype=jnp.int32)
    x_cur = x
    reads = []
    for l in range(3):
        row3 = _interleave_pad(rowc, sent, CHN)
        col3 = _interleave_pad(colc, sent, CHN)
        nmf = nm.astype(jnp.float32)

        dp = deg_pass(ones_cb, row3, col3, zeros_s)
        deg = dp[0, :n] + dp[1, :n] + nmf
        dinv = jnp.where(deg > 0, lax.rsqrt(deg), 0.0)

        Wcat = jnp.concatenate(
            [params['gcn%d_%d_W' % (l, i)] for i in range(heads)], axis=1)
        bcat = jnp.concatenate(
            [params['gcn%d_%d_b' % (l, i)] for i in range(heads)])
        xw = x_cur[:, :sub] @ Wcat
        y = jnp.concatenate(
            [dinv[:, None] * xw, jnp.zeros((R - n, hidden), jnp.float32)])
        ap0 = v_pass(y[:, :half], row3, col3, zeros_v)
        ap1 = v_pass(y[:, half:], row3, col3, zeros_v)
        acc = jnp.concatenate(
            [ap0[0, :n] + ap0[1, :n], ap1[0, :n] + ap1[1, :n]], axis=1)
        subs_cat = jax.nn.relu(
            dinv[:, None] * acc + (nmf * dinv * dinv)[:, None] * xw + bcat)

        sw = jnp.zeros((n,), jnp.float32)
        for i in range(heads):
            si = subs_cat[:, i * sub:(i + 1) * sub]
            a_i = si @ params['att%d_%d_w' % (l, i)]
            t_i = si @ params['score%d_W' % l][i * sub:(i + 1) * sub]
            sw = sw + (a_i * t_i)[:, 0]
        y_s = jnp.concatenate(
            [dinv * sw, jnp.zeros((R - n,), jnp.float32)])
        sp = sc_pass(y_s, row3, col3, zeros_s)
        score = (dinv * (sp[0, :n] + sp[1, :n]) + nmf * dinv * dinv * sw
                 + params['score%d_b' % l][0])

        g = jnp.where(nm, batch, num_graphs).astype(jnp.int32)
        negs = jnp.where(nm, -score, jnp.inf)
        ii = jnp.arange(n, dtype=jnp.int32)
        g_s, _, idx_s = lax.sort((g, negs, ii), num_keys=2)
        cnt = jax.ops.segment_sum(jnp.ones((n,), jnp.int32), g,
                                  num_segments=num_graphs + 1)
        kk = jnp.ceil(ratio * cnt.astype(jnp.float32)).astype(jnp.int32)
        kk = kk.at[num_graphs].set(0)
        starts = jnp.cumsum(cnt) - cnt
        rank = jnp.arange(n, dtype=jnp.int32) - starts[g_s]
        sel_s = rank < kk[g_s]
        sel = jnp.zeros((n,), bool).at[idx_s].set(sel_s)

        xx = subs_cat * jnp.tanh(score)[:, None]
        bs = jnp.where(sel, batch, num_graphs)
        gm = jax.ops.segment_max(xx, bs, num_segments=num_graphs + 1)
        gm = gm[:num_graphs]
        csel = jax.ops.segment_sum(jnp.ones((n,), jnp.float32), bs,
                                   num_segments=num_graphs + 1)[:num_graphs]
        ga = (jax.ops.segment_sum(xx, bs, num_segments=num_graphs + 1)
              [:num_graphs] / jnp.maximum(csel, 1.0)[:, None])
        reads.append(jnp.concatenate([gm, ga], axis=1))
        x_cur = xx
        nm = sel

        if l < 2:
            alive = sel[rowc] & sel[colc] & (ii_e < m)
            m_new = jnp.sum(alive.astype(jnp.int32))
            pos = jnp.cumsum(alive.astype(jnp.int32)) - 1
            dest = jnp.where(alive, pos, sent2)
            gidx = jnp.where(ii_e < m, ii_e, sent2)
            g3 = _interleave_pad(gidx, sent2, CHN)
            d3 = _interleave_pad(dest, sent2, CHN)
            packed = []
            for v in (rowc, colc):
                yv = jnp.concatenate(
                    [v | _BIT, jnp.zeros((R2 - e,), jnp.int32)])
                yv = lax.bitcast_convert_type(yv, jnp.float32)
                pv = pk_pass(yv, g3, d3, zeros_p)
                b0 = lax.bitcast_convert_type(pv[0], jnp.int32)
                b1 = lax.bitcast_convert_type(pv[1], jnp.int32)
                packed.append(jnp.where(b0 != 0, b0, b1)[:e] & ~_BIT)
            rowc = jnp.where(ii_e < m_new, packed[0], sent)
            colc = jnp.where(ii_e < m_new, packed[1], sent)
            m = m_new

    xo = reads[0] + reads[1] + reads[2]
    xo = jax.nn.relu(xo @ params['lin1_W'] + params['lin1_b'])
    xo = jax.nn.relu(xo @ params['lin2_W'] + params['lin2_b'])
    return jax.nn.log_softmax(xo @ params['lin3_W'] + params['lin3_b'],
                              axis=-1)
